# baseline (speedup 1.0000x reference)
import functools
import math

import jax
import jax.numpy as jnp
from jax import lax
from jax.experimental import pallas as pl
from jax.experimental.pallas import tpu as pltpu
from jax.experimental.pallas import tpu_sc as plsc

E = 8
C = 1024
H = 2048
O = 1024
T = 256
CH = 32
NCHUNK = 16
P = NCHUNK * CH
NW = 32
TPW = T // NW
_INV_SQRT2 = 1.0 / math.sqrt(2.0)


def _gate_body(x_ref, wgr_ref, wg_ref, pos_ref, ce_ref, ss_ref):
    x = x_ref[...]
    wgr = wgr_ref[...]
    wg = wg_ref[...]

    xr = lax.dot_general(x, wgr, (((1,), (1,)), ((), ())),
                         preferred_element_type=jnp.float32)
    norm = jnp.sqrt(jnp.sum(wg * wg, axis=1, keepdims=True))
    wg_r = wg * (1.5 / norm)
    n2 = jnp.sqrt(jnp.sum(wg_r * wg_r, axis=1, keepdims=True))
    wg_n = wg_r / jnp.maximum(n2, 1e-4)
    logits = lax.dot_general(xr, wg_n, (((1,), (1,)), ((), ())),
                             preferred_element_type=jnp.float32)

    m = jnp.max(logits, axis=1, keepdims=True)
    ssum = jnp.sum(jnp.exp(logits - m), axis=1, keepdims=True)
    score = 1.0 / ssum

    iota_e = lax.broadcasted_iota(jnp.int32, (T, E), 1)
    idx = jnp.min(jnp.where(logits >= m, iota_e, E), axis=1, keepdims=True)
    onehot = (iota_e == idx).astype(jnp.float32)

    r_i = lax.broadcasted_iota(jnp.int32, (T, T), 0)
    c_i = lax.broadcasted_iota(jnp.int32, (T, T), 1)
    tri = (c_i <= r_i).astype(jnp.float32)
    incl = jnp.dot(tri, onehot, preferred_element_type=jnp.float32)
    rank = jnp.sum(incl * onehot, axis=1, keepdims=True) - 1.0

    counts = jnp.sum(onehot, axis=0, keepdims=True)
    chunks = (counts.astype(jnp.int32) + (CH - 1)) // CH
    er = lax.broadcasted_iota(jnp.int32, (E, E), 0)
    ec = lax.broadcasted_iota(jnp.int32, (E, E), 1)
    tri_s = (er < ec).astype(jnp.float32)
    cstart = jnp.dot(chunks.astype(jnp.float32), tri_s,
                     preferred_element_type=jnp.float32)
    total = jnp.sum(chunks)

    cstart_tok = jnp.dot(onehot, cstart.reshape(E, 1),
                         preferred_element_type=jnp.float32)
    pos = (CH * cstart_tok + rank).astype(jnp.int32)
    pos_ref[...] = pos

    kk = lax.broadcasted_iota(jnp.int32, (NCHUNK, 1), 0)
    kk = jnp.minimum(kk, total - 1)
    ge = (kk >= cstart.astype(jnp.int32)).astype(jnp.int32)
    ce_ref[...] = jnp.sum(ge, axis=1, keepdims=True) - 1

    iota_p = lax.broadcasted_iota(jnp.int32, (T, P), 1)
    sel = (iota_p == pos).astype(jnp.float32)
    ss_ref[...] = lax.dot_general(sel, score, (((0,), (0,)), ((), ())),
                                  preferred_element_type=jnp.float32)


_gate = pl.pallas_call(
    _gate_body,
    out_shape=(
        jax.ShapeDtypeStruct((T, 1), jnp.int32),
        jax.ShapeDtypeStruct((NCHUNK, 1), jnp.int32),
        jax.ShapeDtypeStruct((P, 1), jnp.float32),
    ),
)


@functools.cache
def _make_scatter_x():
    mesh = plsc.VectorSubcoreMesh(core_axis_name="c", subcore_axis_name="s")

    @functools.partial(
        pl.kernel,
        mesh=mesh,
        out_type=jax.ShapeDtypeStruct((P, C), jnp.float32),
        scratch_types=[
            pltpu.VMEM((TPW,), jnp.int32),
            pltpu.VMEM((TPW, C), jnp.float32),
            pltpu.SemaphoreType.DMA,
        ],
        compiler_params=pltpu.CompilerParams(needs_layout_passes=False),
    )
    def _scatter_x(x_hbm, pos_hbm, xs_hbm, idx_v, rows_v, sem):
        wid = lax.axis_index("s") * 2 + lax.axis_index("c")
        base = wid * TPW
        pltpu.sync_copy(pos_hbm.at[pl.ds(base, TPW)], idx_v)
        pltpu.sync_copy(x_hbm.at[pl.ds(base, TPW)], rows_v)
        pltpu.async_copy(rows_v, xs_hbm.at[idx_v], sem).wait()

    return _scatter_x


HH = H // 2


def _ffn_body(ce_ref, xs_ref, w1a_ref, w1b_ref, b1_ref, w2a_ref, w2b_ref,
              b2_ref, ss_ref, ys_ref):
    x = xs_ref[...]
    b1 = b1_ref[0]
    ha = jnp.dot(x, w1a_ref[0], preferred_element_type=jnp.float32)
    ha = ha + b1[:, :HH]
    ha = 0.5 * ha * (1.0 + lax.erf(ha * _INV_SQRT2))
    hb = jnp.dot(x, w1b_ref[0], preferred_element_type=jnp.float32)
    hb = hb + b1[:, HH:]
    hb = 0.5 * hb * (1.0 + lax.erf(hb * _INV_SQRT2))
    y = jnp.dot(ha, w2a_ref[0], preferred_element_type=jnp.float32)
    y = y + jnp.dot(hb, w2b_ref[0], preferred_element_type=jnp.float32)
    ys_ref[...] = (y + b2_ref[0]) * ss_ref[...]


_ffn = pl.pallas_call(
    _ffn_body,
    grid_spec=pltpu.PrefetchScalarGridSpec(
        num_scalar_prefetch=1,
        grid=(NCHUNK,),
        in_specs=[
            pl.BlockSpec((CH, C), lambda c, ce: (c, 0)),
            pl.BlockSpec((1, C, HH), lambda c, ce: (ce[c, 0], 0, 0)),
            pl.BlockSpec((1, C, HH), lambda c, ce: (ce[c, 0], 0, 1)),
            pl.BlockSpec((1, 1, H), lambda c, ce: (ce[c, 0], 0, 0)),
            pl.BlockSpec((1, HH, O), lambda c, ce: (ce[c, 0], 0, 0)),
            pl.BlockSpec((1, HH, O), lambda c, ce: (ce[c, 0], 1, 0)),
            pl.BlockSpec((1, 1, O), lambda c, ce: (ce[c, 0], 0, 0)),
            pl.BlockSpec((CH, 1), lambda c, ce: (c, 0)),
        ],
        out_specs=pl.BlockSpec((CH, O), lambda c, ce: (c, 0)),
    ),
    out_shape=jax.ShapeDtypeStruct((P, O), jnp.float32),
)


@functools.cache
def _make_gather_out():
    mesh = plsc.VectorSubcoreMesh(core_axis_name="c", subcore_axis_name="s")

    @functools.partial(
        pl.kernel,
        mesh=mesh,
        out_type=jax.ShapeDtypeStruct((T, O), jnp.float32),
        scratch_types=[
            pltpu.VMEM((TPW,), jnp.int32),
            pltpu.VMEM((TPW, O), jnp.float32),
            pltpu.SemaphoreType.DMA,
        ],
        compiler_params=pltpu.CompilerParams(needs_layout_passes=False),
    )
    def _gather_out(ys_hbm, pos_hbm, out_hbm, idx_v, rows_v, sem):
        wid = lax.axis_index("s") * 2 + lax.axis_index("c")
        base = wid * TPW
        pltpu.sync_copy(pos_hbm.at[pl.ds(base, TPW)], idx_v)
        pltpu.async_copy(ys_hbm.at[idx_v], rows_v, sem).wait()
        pltpu.sync_copy(rows_v, out_hbm.at[pl.ds(base, TPW)])

    return _gather_out


def kernel(hidden_states, wg_reduction_weight, wg, weight1, bias1, weight2,
           bias2):
    B, S, _ = hidden_states.shape
    x = hidden_states.reshape(T, C)
    pos2, ce2, ss2 = _gate(x, wg_reduction_weight, wg)
    pos = pos2.reshape(T)
    xs = _make_scatter_x()(x, pos)
    ys = _ffn(ce2, xs, weight1, weight1, bias1.reshape(E, 1, H), weight2,
              weight2, bias2.reshape(E, 1, O), ss2)
    out = _make_gather_out()(ys, pos)
    return out.reshape(B, S, O)


# speedup vs baseline: 1.0380x; 1.0380x over previous
import functools
import math

import jax
import jax.numpy as jnp
from jax import lax
from jax.experimental import pallas as pl
from jax.experimental.pallas import tpu as pltpu
from jax.experimental.pallas import tpu_sc as plsc

E = 8
C = 1024
H = 2048
O = 1024
T = 256
CH = 32
NCHUNK = 16
P = NCHUNK * CH
NW = 32
TPW = T // NW
_INV_SQRT2 = 1.0 / math.sqrt(2.0)


def _gate_body(x_ref, wgr_ref, wg_ref, pos_ref, ce_ref, ss_ref, sel_ref):
    x = x_ref[...]
    wgr = wgr_ref[...]
    wg = wg_ref[...]

    xr = lax.dot_general(x, wgr, (((1,), (1,)), ((), ())),
                         preferred_element_type=jnp.float32)
    norm = jnp.sqrt(jnp.sum(wg * wg, axis=1, keepdims=True))
    wg_r = wg * (1.5 / norm)
    n2 = jnp.sqrt(jnp.sum(wg_r * wg_r, axis=1, keepdims=True))
    wg_n = wg_r / jnp.maximum(n2, 1e-4)
    logits = lax.dot_general(xr, wg_n, (((1,), (1,)), ((), ())),
                             preferred_element_type=jnp.float32)

    m = jnp.max(logits, axis=1, keepdims=True)
    ssum = jnp.sum(jnp.exp(logits - m), axis=1, keepdims=True)
    score = 1.0 / ssum

    iota_e = lax.broadcasted_iota(jnp.int32, (T, E), 1)
    idx = jnp.min(jnp.where(logits >= m, iota_e, E), axis=1, keepdims=True)
    onehot = (iota_e == idx).astype(jnp.float32)

    r_i = lax.broadcasted_iota(jnp.int32, (T, T), 0)
    c_i = lax.broadcasted_iota(jnp.int32, (T, T), 1)
    tri = (c_i <= r_i).astype(jnp.float32)
    incl = jnp.dot(tri, onehot, preferred_element_type=jnp.float32)
    rank = jnp.sum(incl * onehot, axis=1, keepdims=True) - 1.0

    counts = jnp.sum(onehot, axis=0, keepdims=True)
    chunks = (counts.astype(jnp.int32) + (CH - 1)) // CH
    er = lax.broadcasted_iota(jnp.int32, (E, E), 0)
    ec = lax.broadcasted_iota(jnp.int32, (E, E), 1)
    tri_s = (er < ec).astype(jnp.float32)
    cstart = jnp.dot(chunks.astype(jnp.float32), tri_s,
                     preferred_element_type=jnp.float32)
    total = jnp.sum(chunks)

    cstart_tok = jnp.dot(onehot, cstart.reshape(E, 1),
                         preferred_element_type=jnp.float32)
    pos = (CH * cstart_tok + rank).astype(jnp.int32)
    pos_ref[...] = pos

    kk = lax.broadcasted_iota(jnp.int32, (NCHUNK, 1), 0)
    kk = jnp.minimum(kk, total - 1)
    ge = (kk >= cstart.astype(jnp.int32)).astype(jnp.int32)
    ce_ref[...] = jnp.sum(ge, axis=1, keepdims=True) - 1

    iota_p = lax.broadcasted_iota(jnp.int32, (T, P), 1)
    sel = (iota_p == pos).astype(jnp.float32)
    sel_ref[...] = sel
    ss_ref[...] = lax.dot_general(sel, score, (((0,), (0,)), ((), ())),
                                  preferred_element_type=jnp.float32)


_gate = pl.pallas_call(
    _gate_body,
    out_shape=(
        jax.ShapeDtypeStruct((T, 1), jnp.int32),
        jax.ShapeDtypeStruct((NCHUNK, 1), jnp.int32),
        jax.ShapeDtypeStruct((P, 1), jnp.float32),
        jax.ShapeDtypeStruct((T, P), jnp.float32),
    ),
)


@functools.cache
def _make_scatter_x():
    mesh = plsc.VectorSubcoreMesh(core_axis_name="c", subcore_axis_name="s")

    @functools.partial(
        pl.kernel,
        mesh=mesh,
        out_type=jax.ShapeDtypeStruct((P, C), jnp.float32),
        scratch_types=[
            pltpu.VMEM((TPW,), jnp.int32),
            pltpu.VMEM((TPW, C), jnp.float32),
            pltpu.SemaphoreType.DMA,
        ],
        compiler_params=pltpu.CompilerParams(needs_layout_passes=False),
    )
    def _scatter_x(x_hbm, pos_hbm, xs_hbm, idx_v, rows_v, sem):
        wid = lax.axis_index("s") * 2 + lax.axis_index("c")
        base = wid * TPW
        pltpu.sync_copy(pos_hbm.at[pl.ds(base, TPW)], idx_v)
        pltpu.sync_copy(x_hbm.at[pl.ds(base, TPW)], rows_v)
        pltpu.async_copy(rows_v, xs_hbm.at[idx_v], sem).wait()

    return _scatter_x


def _ffn_body(ce_ref, xs_ref, w1_ref, b1_ref, w2_ref, b2_ref, ss_ref,
              sel_ref, out_ref, ys_ref):
    c = pl.program_id(0)
    x = xs_ref[...]
    h = jnp.dot(x, w1_ref[0], preferred_element_type=jnp.float32)
    h = h + b1_ref[0]
    h = 0.5 * h * (1.0 + lax.erf(h * _INV_SQRT2))
    y = jnp.dot(h, w2_ref[0], preferred_element_type=jnp.float32)
    ss = ss_ref[...]
    row = pl.multiple_of(c * CH, CH)
    ys_ref[pl.ds(row, CH), :] = jnp.where(ss > 0.0, (y + b2_ref[0]) * ss, 0.0)

    @pl.when(c == NCHUNK - 1)
    def _():
        out_ref[...] = jnp.dot(sel_ref[...], ys_ref[...],
                               preferred_element_type=jnp.float32)


_ffn = pl.pallas_call(
    _ffn_body,
    grid_spec=pltpu.PrefetchScalarGridSpec(
        num_scalar_prefetch=1,
        grid=(NCHUNK,),
        in_specs=[
            pl.BlockSpec((CH, C), lambda c, ce: (c, 0)),
            pl.BlockSpec((1, C, H), lambda c, ce: (ce[c, 0], 0, 0)),
            pl.BlockSpec((1, 1, H), lambda c, ce: (ce[c, 0], 0, 0)),
            pl.BlockSpec((1, H, O), lambda c, ce: (ce[c, 0], 0, 0)),
            pl.BlockSpec((1, 1, O), lambda c, ce: (ce[c, 0], 0, 0)),
            pl.BlockSpec((CH, 1), lambda c, ce: (c, 0)),
            pl.BlockSpec((T, P), lambda c, ce: (0, 0)),
        ],
        out_specs=pl.BlockSpec((T, O), lambda c, ce: (0, 0)),
        scratch_shapes=[pltpu.VMEM((P, O), jnp.float32)],
    ),
    out_shape=jax.ShapeDtypeStruct((T, O), jnp.float32),
)


def kernel(hidden_states, wg_reduction_weight, wg, weight1, bias1, weight2,
           bias2):
    B, S, _ = hidden_states.shape
    x = hidden_states.reshape(T, C)
    pos2, ce2, ss2, sel = _gate(x, wg_reduction_weight, wg)
    pos = pos2.reshape(T)
    xs = _make_scatter_x()(x, pos)
    out = _ffn(ce2, xs, weight1, bias1.reshape(E, 1, H), weight2,
               bias2.reshape(E, 1, O), ss2, sel)
    return out.reshape(B, S, O)
